# baseline (speedup 1.0000x reference)
import functools

import jax
import jax.numpy as jnp
from jax import lax
from jax.experimental import pallas as pl
from jax.experimental.pallas import tpu as pltpu
from jax.experimental.pallas import tpu_sc as plsc

_NC, _NS = 2, 16
_NW = _NC * _NS

_U_ROWS, _I_ROWS, _DIM = 100000, 1000000, 32
_U_CHUNK = 200
_I_CHUNK = 320
_NBUF = 3


def _phase(src, dst, n_rows, chunk, wid, bufs, sin, sout):
    n_chunks = n_rows // chunk
    j_max = -(-n_chunks // _NW)
    n_groups = -(-j_max // _NBUF)

    def pred(j):
        return (j * _NW + wid) < n_chunks

    def base(j):
        return pl.multiple_of((j * _NW + wid) * chunk, 8)

    def in_copy(j, b):
        return pltpu.make_async_copy(
            src.at[pl.ds(base(j), chunk)], bufs[b].at[pl.ds(0, chunk)], sin[b])

    def out_copy(j, b):
        return pltpu.make_async_copy(
            bufs[b].at[pl.ds(0, chunk)], dst.at[pl.ds(base(j), chunk)], sout[b])

    def group(g, carry):
        for b in range(_NBUF):
            j = g * _NBUF + b
            pl.when(pred(j))(in_copy(j, b).start)
        for b in range(_NBUF):
            j = g * _NBUF + b

            @pl.when(pred(j))
            def _():
                in_copy(j, b).wait()
                out_copy(j, b).start()
        for b in range(_NBUF):
            j = g * _NBUF + b
            pl.when(pred(j))(out_copy(j, b).wait)
        return carry

    lax.fori_loop(0, n_groups, group, 0)


def _sc_copy_body(u_in, i_in, u_out, i_out, *scratch):
    wid = lax.axis_index("s") * _NC + lax.axis_index("c")
    bufs = scratch[:_NBUF]
    sin = scratch[_NBUF:2 * _NBUF]
    sout = scratch[2 * _NBUF:]
    _phase(i_in, i_out, _I_ROWS, _I_CHUNK, wid, bufs, sin, sout)
    _phase(u_in, u_out, _U_ROWS, _U_CHUNK, wid, bufs, sin, sout)


@functools.partial(
    pl.kernel,
    out_type=(
        jax.ShapeDtypeStruct((_U_ROWS, _DIM), jnp.float32),
        jax.ShapeDtypeStruct((_I_ROWS, _DIM), jnp.float32),
    ),
    mesh=plsc.VectorSubcoreMesh(core_axis_name="c", subcore_axis_name="s"),
    compiler_params=pltpu.CompilerParams(use_tc_tiling_on_sc=False),
    scratch_types=(
        [pltpu.VMEM((_I_CHUNK, _DIM), jnp.float32)] * _NBUF
        + [pltpu.SemaphoreType.DMA] * (2 * _NBUF)
    ),
)
def _sc_copy(u_in, i_in, u_out, i_out, *scratch):
    _sc_copy_body(u_in, i_in, u_out, i_out, *scratch)


def kernel(user_weight, item_weight, edge_index):
    return _sc_copy(user_weight, item_weight)


# speedup vs baseline: 1.1384x; 1.1384x over previous
import functools

import jax
import jax.numpy as jnp
from jax import lax
from jax.experimental import pallas as pl
from jax.experimental.pallas import tpu as pltpu
from jax.experimental.pallas import tpu_sc as plsc

_NC, _NS = 2, 16
_NW = _NC * _NS

_U_ROWS, _I_ROWS, _DIM = 100000, 1000000, 32
_U_CHUNK = 200
_I_CHUNK = 320
_NBUF = 3


def _phase(src, dst, n_rows, chunk, wid, bufs, sin, sout):
    n_chunks = n_rows // chunk
    j_max = -(-n_chunks // _NW)
    n_groups = -(-j_max // _NBUF)

    def pred(j):
        return (j * _NW + wid) < n_chunks

    def base(j):
        return pl.multiple_of((j * _NW + wid) * chunk, 8)

    def in_copy(j, b):
        return pltpu.make_async_copy(
            src.at[pl.ds(base(j), chunk)], bufs[b].at[pl.ds(0, chunk)], sin[b])

    def out_copy(j, b):
        return pltpu.make_async_copy(
            bufs[b].at[pl.ds(0, chunk)], dst.at[pl.ds(base(j), chunk)], sout[b])

    def group(g, carry):
        for b in range(_NBUF):
            j = g * _NBUF + b
            pl.when(pred(j))(in_copy(j, b).start)
        for b in range(_NBUF):
            j = g * _NBUF + b

            @pl.when(pred(j))
            def _():
                in_copy(j, b).wait()
                out_copy(j, b).start()
        for b in range(_NBUF):
            j = g * _NBUF + b
            pl.when(pred(j))(out_copy(j, b).wait)
        return carry

    lax.fori_loop(0, n_groups, group, 0)


def _sc_copy_body(u_in, i_in, u_out, i_out, *scratch):
    wid = lax.axis_index("s") * _NC + lax.axis_index("c")
    bufs = scratch[:_NBUF]
    sin = scratch[_NBUF:2 * _NBUF]
    sout = scratch[2 * _NBUF:]
    _phase(i_in, i_out, _I_ROWS, _I_CHUNK, wid, bufs, sin, sout)
    _phase(u_in, u_out, _U_ROWS, _U_CHUNK, wid, bufs, sin, sout)


@functools.partial(
    pl.kernel,
    out_type=(
        jax.ShapeDtypeStruct((_U_ROWS, _DIM), jnp.float32),
        jax.ShapeDtypeStruct((_I_ROWS, _DIM), jnp.float32),
    ),
    mesh=plsc.VectorSubcoreMesh(core_axis_name="c", subcore_axis_name="s"),
    scratch_types=(
        [pltpu.VMEM((_I_CHUNK, _DIM), jnp.float32)] * _NBUF
        + [pltpu.SemaphoreType.DMA] * (2 * _NBUF)
    ),
)
def _sc_copy(u_in, i_in, u_out, i_out, *scratch):
    _sc_copy_body(u_in, i_in, u_out, i_out, *scratch)


def kernel(user_weight, item_weight, edge_index):
    return _sc_copy(user_weight, item_weight)


# speedup vs baseline: 13.3411x; 11.7190x over previous
import jax
import jax.numpy as jnp
from jax.experimental import pallas as pl
from jax.experimental.pallas import tpu as pltpu

_U_ROWS, _I_ROWS, _DIM = 100000, 1000000, 32
_GRID = 32
_U_W = 3200
_I_W = 32000


def _copy_body(u_in, i_in, u_out, i_out):
    u_out[...] = u_in[...]
    i_out[...] = i_in[...]


def kernel(user_weight, item_weight, edge_index):
    u_t = user_weight.T
    i_t = item_weight.T
    out_shape = (
        jax.ShapeDtypeStruct(u_t.shape, u_t.dtype),
        jax.ShapeDtypeStruct(i_t.shape, i_t.dtype),
    )
    uo, io = pl.pallas_call(
        _copy_body,
        grid=(_GRID,),
        in_specs=[
            pl.BlockSpec((_DIM, _U_W), lambda g: (0, g)),
            pl.BlockSpec((_DIM, _I_W), lambda g: (0, g)),
        ],
        out_specs=(
            pl.BlockSpec((_DIM, _U_W), lambda g: (0, g)),
            pl.BlockSpec((_DIM, _I_W), lambda g: (0, g)),
        ),
        out_shape=out_shape,
        compiler_params=pltpu.CompilerParams(
            dimension_semantics=("arbitrary",),
        ),
    )(u_t, i_t)
    return uo.T, io.T


# speedup vs baseline: 13.6396x; 1.0224x over previous
import jax
import jax.numpy as jnp
from jax.experimental import pallas as pl
from jax.experimental.pallas import tpu as pltpu

_U_ROWS, _I_ROWS, _DIM = 100000, 1000000, 32
_GRID = 16
_U_W = 6400
_I_W = 64000


def _copy_body(u_in, i_in, u_out, i_out):
    u_out[...] = u_in[...]
    i_out[...] = i_in[...]


def kernel(user_weight, item_weight, edge_index):
    u_t = user_weight.T
    i_t = item_weight.T
    out_shape = (
        jax.ShapeDtypeStruct(u_t.shape, u_t.dtype),
        jax.ShapeDtypeStruct(i_t.shape, i_t.dtype),
    )
    uo, io = pl.pallas_call(
        _copy_body,
        grid=(_GRID,),
        in_specs=[
            pl.BlockSpec((_DIM, _U_W), lambda g: (0, g)),
            pl.BlockSpec((_DIM, _I_W), lambda g: (0, g)),
        ],
        out_specs=(
            pl.BlockSpec((_DIM, _U_W), lambda g: (0, g)),
            pl.BlockSpec((_DIM, _I_W), lambda g: (0, g)),
        ),
        out_shape=out_shape,
        compiler_params=pltpu.CompilerParams(
            dimension_semantics=("arbitrary",),
        ),
    )(u_t, i_t)
    return uo.T, io.T
